# Initial kernel scaffold; baseline (speedup 1.0000x reference)
import jax
import jax.numpy as jnp
from jax.experimental import pallas as pl


def kernel(voc_features, sms_features, personal_feature, edge_index0, edge_index1, edge_index2, lstm_voc_Wih, lstm_voc_Whh, lstm_voc_bih, lstm_voc_bhh, lstm_sms_Wih, lstm_sms_Whh, lstm_sms_bih, lstm_sms_bhh, lin_voc_W, lin_voc_b, lin_sms_W, lin_sms_b, lin_per_W, lin_per_b, lin2_W, lin2_b, lin3_W, lin3_b, lin4_W, lin4_b, Wf1_0, bf1_0, Wf2_0, Wf1_1, bf1_1, Wf2_1, Wf1_2, bf1_2, Wf2_2, lin5_W0, lin5_b0, lin5_W1, lin5_b1, lin5_W2, lin5_b2, lin6_W, lin6_b):
    raise NotImplementedError("write your pallas kernel here")


# speedup vs baseline: 6.8768x; 6.8768x over previous
import functools

import jax
import jax.numpy as jnp
from jax import lax
from jax.experimental import pallas as pl
from jax.experimental.pallas import tpu as pltpu, tpu_sc as plsc

N = 10000
T = 20
DV = 32
DS = 32
DP = 16
HH = 32
H = 64
C = 2
E = 640000

NPAD = 10240
NC = 2
NS = 16
NW = NC * NS
EW = E // NW
CH = 80
NCHUNK = EW // CH

BN = 1024
GRID = NPAD // BN

_RPT = NPAD // NS


def _sc_pass_body(y_hbm, src_hbm, dst_hbm, zeros_hbm, out_hbm,
                  src_v, dst_v, rows_v, acc_sh, sem):
    c = lax.axis_index("c")
    s = lax.axis_index("s")
    wid = s * NC + c
    pltpu.sync_copy(src_hbm.at[wid], src_v)
    pltpu.sync_copy(dst_hbm.at[wid], dst_v)
    pltpu.sync_copy(zeros_hbm.at[pl.ds(s * _RPT, _RPT)],
                    acc_sh.at[pl.ds(s * _RPT, _RPT)])
    plsc.subcore_barrier()

    def body(j, carry):
        pltpu.async_copy(y_hbm.at[src_v.at[j]], rows_v, sem).wait()
        pltpu.sync_copy(rows_v, acc_sh.at[dst_v.at[j]], add=True)
        return carry

    lax.fori_loop(0, NCHUNK, body, 0)
    plsc.subcore_barrier()
    pltpu.sync_copy(acc_sh.at[pl.ds(s * _RPT, _RPT)],
                    out_hbm.at[c, pl.ds(s * _RPT, _RPT)])


def _sc_deg_body(dst0_hbm, dst1_hbm, dst2_hbm, zeros1_hbm, out_hbm,
                 idx_v, ones_v, deg0_sh, deg1_sh, deg2_sh):
    c = lax.axis_index("c")
    s = lax.axis_index("s")
    wid = s * NC + c
    for k in range(CH // 16):
        ones_v[pl.ds(k * 16, 16)] = jnp.ones((16,), jnp.float32)
    degs = (deg0_sh, deg1_sh, deg2_sh)
    for d_sh in degs:
        pltpu.sync_copy(zeros1_hbm.at[pl.ds(s * _RPT, _RPT)],
                        d_sh.at[pl.ds(s * _RPT, _RPT)])
    plsc.subcore_barrier()
    for dst_hbm, d_sh in zip((dst0_hbm, dst1_hbm, dst2_hbm), degs):
        pltpu.sync_copy(dst_hbm.at[wid], idx_v)

        def body(j, carry, d_sh=d_sh):
            pltpu.sync_copy(ones_v, d_sh.at[idx_v.at[j]], add=True)
            return carry

        lax.fori_loop(0, NCHUNK, body, 0)
    plsc.subcore_barrier()
    for r, d_sh in enumerate(degs):
        pltpu.sync_copy(d_sh.at[pl.ds(s * _RPT, _RPT)],
                        out_hbm.at[c, 0, pl.ds(r * NPAD + s * _RPT, _RPT)])


@functools.lru_cache(maxsize=None)
def _build_sc_kernels():
    mesh = plsc.VectorSubcoreMesh(core_axis_name="c", subcore_axis_name="s")
    params = pltpu.CompilerParams(use_tc_tiling_on_sc=False)
    sc_pass = pl.kernel(
        _sc_pass_body,
        out_type=jax.ShapeDtypeStruct((NC, NPAD, H), jnp.float32),
        mesh=mesh,
        compiler_params=params,
        scratch_types=[
            pltpu.VMEM((NCHUNK, CH), jnp.int32),
            pltpu.VMEM((NCHUNK, CH), jnp.int32),
            pltpu.VMEM((CH, H), jnp.float32),
            pltpu.VMEM_SHARED((NPAD, H), jnp.float32),
            pltpu.SemaphoreType.DMA,
        ],
    )
    sc_deg = pl.kernel(
        _sc_deg_body,
        out_type=jax.ShapeDtypeStruct((NC, 1, 3 * NPAD), jnp.float32),
        mesh=mesh,
        compiler_params=params,
        scratch_types=[
            pltpu.VMEM((NCHUNK, CH), jnp.int32),
            pltpu.VMEM((CH,), jnp.float32),
            pltpu.VMEM_SHARED((NPAD,), jnp.float32),
            pltpu.VMEM_SHARED((NPAD,), jnp.float32),
            pltpu.VMEM_SHARED((NPAD,), jnp.float32),
        ],
    )
    return sc_pass, sc_deg


def _segment_partials(y, src_w, dst_w, zeros_h):
    return _build_sc_kernels()[0](y, src_w, dst_w, zeros_h)


def _deg_partials(dst0_w, dst1_w, dst2_w, zeros1):
    return _build_sc_kernels()[1](dst0_w, dst1_w, dst2_w, zeros1)


def _full(shape):
    return pl.BlockSpec(shape, lambda i: (0,) * len(shape))


def _lstm_scan(x_ref, WiT, WhT, b):
    h0 = jnp.zeros((BN, HH), jnp.float32)
    c0 = jnp.zeros((BN, HH), jnp.float32)

    def step(t, hc):
        h, c = hc
        xt = x_ref[t]
        g = jnp.dot(xt, WiT, preferred_element_type=jnp.float32)
        g = g + jnp.dot(h, WhT, preferred_element_type=jnp.float32) + b
        i = jax.nn.sigmoid(g[:, 0 * HH:1 * HH])
        f = jax.nn.sigmoid(g[:, 1 * HH:2 * HH])
        gg = jnp.tanh(g[:, 2 * HH:3 * HH])
        o = jax.nn.sigmoid(g[:, 3 * HH:4 * HH])
        c2 = f * c + i * gg
        h2 = o * jnp.tanh(c2)
        return (h2, c2)

    h, _ = lax.fori_loop(0, T, step, (h0, c0))
    return h


def _leaky(x):
    return jnp.where(x >= 0, x, 0.01 * x)


def _encoder_body(xvt_ref, xst_ref, per_ref, degp_ref,
                  WivT, WhvT, bv, WisT, WhsT, bs,
                  LvT, lbv, LsT, lbs, LpT, lbp,
                  W2T, b2, W3T, b3, W4T, b4,
                  x0_o, x1_o, x2_o, y0_o, y1_o, y2_o, dinv_o):
    hv = _lstm_scan(xvt_ref, WivT[:], WhvT[:], bv[:])
    hs = _lstm_scan(xst_ref, WisT[:], WhsT[:], bs[:])
    xv = jnp.dot(hv, LvT[:], preferred_element_type=jnp.float32) + lbv[:]
    xs = jnp.dot(hs, LsT[:], preferred_element_type=jnp.float32) + lbs[:]
    xp = jnp.dot(per_ref[:], LpT[:], preferred_element_type=jnp.float32) + lbp[:]
    xv2 = _leaky(jnp.concatenate([xv, xp], axis=1))
    xs2 = _leaky(jnp.concatenate([xs, xp], axis=1))
    xv3 = _leaky(jnp.dot(xv2, W2T[:], preferred_element_type=jnp.float32) + b2[:])
    xs3 = _leaky(jnp.dot(xs2, W3T[:], preferred_element_type=jnp.float32) + b3[:])
    xvs = jnp.concatenate([xv2, xs2], axis=1)
    xvs3 = _leaky(jnp.dot(xvs, W4T[:], preferred_element_type=jnp.float32) + b4[:])
    x0_o[:] = xv3
    x1_o[:] = xs3
    x2_o[:] = xvs3
    xins = (xv3, xs3, xvs3)
    youts = (y0_o, y1_o, y2_o)
    for r in range(3):
        deg = degp_ref[0, r] + degp_ref[1, r]
        dinv = lax.rsqrt(jnp.maximum(deg, 1.0))
        dinv_o[r] = dinv
        youts[r][:] = xins[r] * dinv


def _encoder(xvt, xst, per, degp, weights):
    outs = (
        [jax.ShapeDtypeStruct((NPAD, H), jnp.float32)] * 6
        + [jax.ShapeDtypeStruct((3, NPAD, 1), jnp.float32)]
    )
    out_specs = (
        [pl.BlockSpec((BN, H), lambda i: (i, 0))] * 6
        + [pl.BlockSpec((3, BN, 1), lambda i: (0, i, 0))]
    )
    in_specs = [
        pl.BlockSpec((T, BN, DV), lambda i: (0, i, 0)),
        pl.BlockSpec((T, BN, DS), lambda i: (0, i, 0)),
        pl.BlockSpec((BN, DP), lambda i: (i, 0)),
        pl.BlockSpec((NC, 3, BN, 1), lambda i: (0, 0, i, 0)),
    ] + [_full(w.shape) for w in weights]
    return pl.pallas_call(
        _encoder_body,
        grid=(GRID,),
        in_specs=in_specs,
        out_specs=out_specs,
        out_shape=outs,
    )(xvt, xst, per, degp, *weights)


def _update_body(x_ref, p_ref, dinv_ref, f_o, y_o):
    agg = p_ref[0] + p_ref[1]
    dinv = dinv_ref[:]
    f = x_ref[:] - agg * dinv
    f_o[:] = f
    y_o[:] = f * dinv


def _update(x, partials, dinv_r):
    return pl.pallas_call(
        _update_body,
        grid=(GRID,),
        in_specs=[
            pl.BlockSpec((BN, H), lambda i: (i, 0)),
            pl.BlockSpec((NC, BN, H), lambda i: (0, i, 0)),
            pl.BlockSpec((BN, 1), lambda i: (i, 0)),
        ],
        out_specs=[pl.BlockSpec((BN, H), lambda i: (i, 0))] * 2,
        out_shape=[jax.ShapeDtypeStruct((NPAD, H), jnp.float32)] * 2,
    )(x, partials, dinv_r)


_THETA_W = (
    (0.8, -0.5, 0.0),
    (3.0, -3.0, 0.75),
    (0.0, 3.0, -1.5),
    (0.0, 0.0, 0.75),
    (-0.2, 0.5, 0.0),
)


def _attn_body(f0_ref, f1_ref, f2_ref, Wf1T, bf1, wf2, W5T, b5, out_o):
    f0, f1, f2 = f0_ref[:], f1_ref[:], f2_ref[:]
    hs = [t0 * f0 + t1 * f1 + t2 * f2 for (t0, t1, t2) in _THETA_W]
    ps = []
    for hk in hs:
        sk = jnp.tanh(jnp.dot(hk, Wf1T[:], preferred_element_type=jnp.float32)
                      + bf1[:])
        ps.append(jnp.sum(sk * wf2[:], axis=1, keepdims=True))
    m = ps[0]
    for pk in ps[1:]:
        m = jnp.maximum(m, pk)
    es = [jnp.exp(pk - m) for pk in ps]
    z = es[0]
    for ek in es[1:]:
        z = z + ek
    inv_z = 1.0 / z
    res = jnp.zeros((BN, H), jnp.float32)
    for ek, hk in zip(es, hs):
        res = res + (ek * inv_z) * hk
    out_o[:] = jnp.dot(res, W5T[:], preferred_element_type=jnp.float32) + b5[:]


def _attention(f0, f1, f2, Wf1T, bf1, wf2, W5T, b5):
    return pl.pallas_call(
        _attn_body,
        grid=(GRID,),
        in_specs=[pl.BlockSpec((BN, H), lambda i: (i, 0))] * 3
        + [_full(Wf1T.shape), _full(bf1.shape), _full(wf2.shape),
           _full(W5T.shape), _full(b5.shape)],
        out_specs=pl.BlockSpec((BN, H), lambda i: (i, 0)),
        out_shape=jax.ShapeDtypeStruct((NPAD, H), jnp.float32),
    )(f0, f1, f2, Wf1T, bf1, wf2, W5T, b5)


def _final_body(h0_ref, h1_ref, h2_ref, x0_ref, x1_ref, x2_ref, W6T, b6, out_o):
    hcat = _leaky(jnp.concatenate([h0_ref[:], h1_ref[:], h2_ref[:]], axis=1))
    full = jnp.concatenate([hcat, x0_ref[:], x1_ref[:], x2_ref[:]], axis=1)
    out_o[:] = jnp.dot(full, W6T[:], preferred_element_type=jnp.float32) + b6[:]


def _final(h0, h1, h2, x0, x1, x2, W6T, b6):
    return pl.pallas_call(
        _final_body,
        grid=(GRID,),
        in_specs=[pl.BlockSpec((BN, H), lambda i: (i, 0))] * 6
        + [_full(W6T.shape), _full(b6.shape)],
        out_specs=pl.BlockSpec((BN, 128), lambda i: (i, 0)),
        out_shape=jax.ShapeDtypeStruct((NPAD, 128), jnp.float32),
    )(h0, h1, h2, x0, x1, x2, W6T, b6)


def _pad_nodes(x):
    return jnp.pad(x, ((0, NPAD - N),) + ((0, 0),) * (x.ndim - 1))


def kernel(voc_features, sms_features, personal_feature,
           edge_index0, edge_index1, edge_index2,
           lstm_voc_Wih, lstm_voc_Whh, lstm_voc_bih, lstm_voc_bhh,
           lstm_sms_Wih, lstm_sms_Whh, lstm_sms_bih, lstm_sms_bhh,
           lin_voc_W, lin_voc_b, lin_sms_W, lin_sms_b, lin_per_W, lin_per_b,
           lin2_W, lin2_b, lin3_W, lin3_b, lin4_W, lin4_b,
           Wf1_0, bf1_0, Wf2_0, Wf1_1, bf1_1, Wf2_1, Wf1_2, bf1_2, Wf2_2,
           lin5_W0, lin5_b0, lin5_W1, lin5_b1, lin5_W2, lin5_b2,
           lin6_W, lin6_b):
    f32 = jnp.float32
    xvt = _pad_axis1(jnp.swapaxes(voc_features, 0, 1))
    xst = _pad_axis1(jnp.swapaxes(sms_features, 0, 1))
    per = _pad_nodes(personal_feature)
    srcs, dsts = [], []
    for e in (edge_index0, edge_index1, edge_index2):
        srcs.append(e[0].reshape(NW, NCHUNK, CH))
        dsts.append(e[1].reshape(NW, NCHUNK, CH))
    zeros_h = jnp.zeros((NPAD, H), f32)
    zeros1 = jnp.zeros((NPAD,), f32)

    enc_w = [
        lstm_voc_Wih.T, lstm_voc_Whh.T,
        (lstm_voc_bih + lstm_voc_bhh).reshape(1, 4 * HH),
        lstm_sms_Wih.T, lstm_sms_Whh.T,
        (lstm_sms_bih + lstm_sms_bhh).reshape(1, 4 * HH),
        lin_voc_W.T, lin_voc_b.reshape(1, H),
        lin_sms_W.T, lin_sms_b.reshape(1, H),
        lin_per_W.T, lin_per_b.reshape(1, H),
        lin2_W.T, lin2_b.reshape(1, H),
        lin3_W.T, lin3_b.reshape(1, H),
        lin4_W.T, lin4_b.reshape(1, H),
    ]

    degp = _deg_partials(dsts[0], dsts[1], dsts[2], zeros1)
    degp = degp.reshape(NC, 3, NPAD, 1)

    x0, x1, x2, y00, y01, y02, dinv = _encoder(xvt, xst, per, degp, enc_w)
    xins = (x0, x1, x2)
    y0s = (y00, y01, y02)

    att_w = (
        (Wf1_0.T, bf1_0.reshape(1, H), Wf2_0, lin5_W0.T, lin5_b0.reshape(1, H)),
        (Wf1_1.T, bf1_1.reshape(1, H), Wf2_1, lin5_W1.T, lin5_b1.reshape(1, H)),
        (Wf1_2.T, bf1_2.reshape(1, H), Wf2_2, lin5_W2.T, lin5_b2.reshape(1, H)),
    )

    hs = []
    for r in range(3):
        dinv_r = dinv[r]
        p1 = _segment_partials(y0s[r], srcs[r], dsts[r], zeros_h)
        f1, y1 = _update(xins[r], p1, dinv_r)
        p2 = _segment_partials(y1, srcs[r], dsts[r], zeros_h)
        f2, _ = _update(f1, p2, dinv_r)
        hs.append(_attention(xins[r], f1, f2, *att_w[r]))

    W6T = jnp.zeros((6 * H, 128), f32).at[:, :C].set(lin6_W.T)
    b6 = jnp.zeros((1, 128), f32).at[0, :C].set(lin6_b)
    out = _final(hs[0], hs[1], hs[2], x0, x1, x2, W6T, b6)
    return out[:N, :C]


def _pad_axis1(x):
    return jnp.pad(x, ((0, 0), (0, NPAD - N), (0, 0)))


# speedup vs baseline: 10.2485x; 1.4903x over previous
import functools

import jax
import jax.numpy as jnp
from jax import lax
from jax.experimental import pallas as pl
from jax.experimental.pallas import tpu as pltpu, tpu_sc as plsc

N = 10000
T = 20
DV = 32
DS = 32
DP = 16
HH = 32
H = 64
C = 2
E = 640000

NPAD = 10240
NC = 2
NS = 16
NW = NC * NS
EW = E // NW
CH = 80
NCHUNK = EW // CH

BN = 1024
GRID = NPAD // BN

_RPT = NPAD // NS


def _sc_pass_body(y_hbm, src_hbm, dst_hbm, zeros_hbm, out_hbm,
                  src_v, dst_v, rows_a, rows_b, acc_sh, sem_a, sem_b):
    c = lax.axis_index("c")
    s = lax.axis_index("s")
    wid = s * NC + c
    pltpu.sync_copy(src_hbm.at[wid], src_v)
    pltpu.sync_copy(dst_hbm.at[wid], dst_v)
    pltpu.sync_copy(zeros_hbm.at[pl.ds(s * _RPT, _RPT)],
                    acc_sh.at[pl.ds(s * _RPT, _RPT)])
    plsc.subcore_barrier()

    pltpu.async_copy(y_hbm.at[src_v.at[0]], rows_a, sem_a)

    def body(jj, carry):
        j = 2 * jj
        pltpu.async_copy(y_hbm.at[src_v.at[j + 1]], rows_b, sem_b)
        pltpu.make_async_copy(y_hbm.at[src_v.at[j]], rows_a, sem_a).wait()
        pltpu.sync_copy(rows_a, acc_sh.at[dst_v.at[j]], add=True)

        @pl.when(j + 2 < NCHUNK)
        def _():
            pltpu.async_copy(y_hbm.at[src_v.at[j + 2]], rows_a, sem_a)

        pltpu.make_async_copy(y_hbm.at[src_v.at[j + 1]], rows_b, sem_b).wait()
        pltpu.sync_copy(rows_b, acc_sh.at[dst_v.at[j + 1]], add=True)
        return carry

    lax.fori_loop(0, NCHUNK // 2, body, 0)
    plsc.subcore_barrier()
    pltpu.sync_copy(acc_sh.at[pl.ds(s * _RPT, _RPT)],
                    out_hbm.at[c, pl.ds(s * _RPT, _RPT)])


def _sc_deg_body(dst0_hbm, dst1_hbm, dst2_hbm, zeros1_hbm, out_hbm,
                 idx_v, deg0_v, deg1_v, deg2_v):
    c = lax.axis_index("c")
    s = lax.axis_index("s")
    wid = s * NC + c
    degs = (deg0_v, deg1_v, deg2_v)
    for d_v in degs:
        pltpu.sync_copy(zeros1_hbm, d_v)
    ones16 = jnp.ones((16,), jnp.float32)
    for dst_hbm, d_v in zip((dst0_hbm, dst1_hbm, dst2_hbm), degs):
        pltpu.sync_copy(dst_hbm.at[wid, 0], idx_v)

        def body(k, carry, d_v=d_v):
            idx16 = idx_v[pl.ds(k * 16, 16)]
            plsc.addupdate_scatter(d_v, [idx16], ones16)
            return carry

        lax.fori_loop(0, EW // 16, body, 0)
    for r, d_v in enumerate(degs):
        pltpu.sync_copy(d_v, out_hbm.at[wid, 0, pl.ds(r * NPAD, NPAD)])


@functools.lru_cache(maxsize=None)
def _build_sc_kernels():
    mesh = plsc.VectorSubcoreMesh(core_axis_name="c", subcore_axis_name="s")
    params = pltpu.CompilerParams(use_tc_tiling_on_sc=False,
                                  needs_layout_passes=False)
    sc_pass = pl.kernel(
        _sc_pass_body,
        out_type=jax.ShapeDtypeStruct((NC, NPAD, H), jnp.float32),
        mesh=mesh,
        compiler_params=params,
        scratch_types=[
            pltpu.VMEM((NCHUNK, CH), jnp.int32),
            pltpu.VMEM((NCHUNK, CH), jnp.int32),
            pltpu.VMEM((CH, H), jnp.float32),
            pltpu.VMEM((CH, H), jnp.float32),
            pltpu.VMEM_SHARED((NPAD, H), jnp.float32),
            pltpu.SemaphoreType.DMA,
            pltpu.SemaphoreType.DMA,
        ],
    )
    sc_deg = pl.kernel(
        _sc_deg_body,
        out_type=jax.ShapeDtypeStruct((NW, 1, 3 * NPAD), jnp.float32),
        mesh=mesh,
        compiler_params=params,
        scratch_types=[
            pltpu.VMEM((EW,), jnp.int32),
            pltpu.VMEM((NPAD,), jnp.float32),
            pltpu.VMEM((NPAD,), jnp.float32),
            pltpu.VMEM((NPAD,), jnp.float32),
        ],
    )
    return sc_pass, sc_deg


def _segment_partials(y, src_w, dst_w, zeros_h):
    return _build_sc_kernels()[0](y, src_w, dst_w, zeros_h)


def _deg_partials(dst0_w, dst1_w, dst2_w, zeros1):
    return _build_sc_kernels()[1](dst0_w, dst1_w, dst2_w, zeros1)


def _full(shape):
    return pl.BlockSpec(shape, lambda i: (0,) * len(shape))


def _lstm_scan(x_ref, WiT, WhT, b):
    h0 = jnp.zeros((BN, HH), jnp.float32)
    c0 = jnp.zeros((BN, HH), jnp.float32)

    def step(t, hc):
        h, c = hc
        xt = x_ref[t]
        g = jnp.dot(xt, WiT, preferred_element_type=jnp.float32)
        g = g + jnp.dot(h, WhT, preferred_element_type=jnp.float32) + b
        i = jax.nn.sigmoid(g[:, 0 * HH:1 * HH])
        f = jax.nn.sigmoid(g[:, 1 * HH:2 * HH])
        gg = jnp.tanh(g[:, 2 * HH:3 * HH])
        o = jax.nn.sigmoid(g[:, 3 * HH:4 * HH])
        c2 = f * c + i * gg
        h2 = o * jnp.tanh(c2)
        return (h2, c2)

    h, _ = lax.fori_loop(0, T, step, (h0, c0))
    return h


def _leaky(x):
    return jnp.where(x >= 0, x, 0.01 * x)


def _encoder_body(xvt_ref, xst_ref, per_ref, degp_ref,
                  WivT, WhvT, bv, WisT, WhsT, bs,
                  LvT, lbv, LsT, lbs, LpT, lbp,
                  W2T, b2, W3T, b3, W4T, b4,
                  x0_o, x1_o, x2_o, y0_o, y1_o, y2_o, dinv_o):
    hv = _lstm_scan(xvt_ref, WivT[:], WhvT[:], bv[:])
    hs = _lstm_scan(xst_ref, WisT[:], WhsT[:], bs[:])
    xv = jnp.dot(hv, LvT[:], preferred_element_type=jnp.float32) + lbv[:]
    xs = jnp.dot(hs, LsT[:], preferred_element_type=jnp.float32) + lbs[:]
    xp = jnp.dot(per_ref[:], LpT[:], preferred_element_type=jnp.float32) + lbp[:]
    xv2 = _leaky(jnp.concatenate([xv, xp], axis=1))
    xs2 = _leaky(jnp.concatenate([xs, xp], axis=1))
    xv3 = _leaky(jnp.dot(xv2, W2T[:], preferred_element_type=jnp.float32) + b2[:])
    xs3 = _leaky(jnp.dot(xs2, W3T[:], preferred_element_type=jnp.float32) + b3[:])
    xvs = jnp.concatenate([xv2, xs2], axis=1)
    xvs3 = _leaky(jnp.dot(xvs, W4T[:], preferred_element_type=jnp.float32) + b4[:])
    x0_o[:] = xv3
    x1_o[:] = xs3
    x2_o[:] = xvs3
    xins = (xv3, xs3, xvs3)
    youts = (y0_o, y1_o, y2_o)
    deg3 = jnp.sum(degp_ref[:], axis=0)
    for r in range(3):
        dinv = lax.rsqrt(jnp.maximum(deg3[r], 1.0)).reshape(BN, 1)
        dinv_o[r] = dinv
        youts[r][:] = xins[r] * dinv


def _encoder(xvt, xst, per, degp, weights):
    outs = (
        [jax.ShapeDtypeStruct((NPAD, H), jnp.float32)] * 6
        + [jax.ShapeDtypeStruct((3, NPAD, 1), jnp.float32)]
    )
    out_specs = (
        [pl.BlockSpec((BN, H), lambda i: (i, 0))] * 6
        + [pl.BlockSpec((3, BN, 1), lambda i: (0, i, 0))]
    )
    in_specs = [
        pl.BlockSpec((T, BN, DV), lambda i: (0, i, 0)),
        pl.BlockSpec((T, BN, DS), lambda i: (0, i, 0)),
        pl.BlockSpec((BN, DP), lambda i: (i, 0)),
        pl.BlockSpec((NW, 3, BN), lambda i: (0, 0, i)),
    ] + [_full(w.shape) for w in weights]
    return pl.pallas_call(
        _encoder_body,
        grid=(GRID,),
        in_specs=in_specs,
        out_specs=out_specs,
        out_shape=outs,
    )(xvt, xst, per, degp, *weights)


def _update_body(x_ref, p_ref, dinv_ref, f_o, y_o):
    agg = p_ref[0] + p_ref[1]
    dinv = dinv_ref[:]
    f = x_ref[:] - agg * dinv
    f_o[:] = f
    y_o[:] = f * dinv


def _update(x, partials, dinv_r):
    return pl.pallas_call(
        _update_body,
        grid=(GRID,),
        in_specs=[
            pl.BlockSpec((BN, H), lambda i: (i, 0)),
            pl.BlockSpec((NC, BN, H), lambda i: (0, i, 0)),
            pl.BlockSpec((BN, 1), lambda i: (i, 0)),
        ],
        out_specs=[pl.BlockSpec((BN, H), lambda i: (i, 0))] * 2,
        out_shape=[jax.ShapeDtypeStruct((NPAD, H), jnp.float32)] * 2,
    )(x, partials, dinv_r)


_THETA_W = (
    (0.8, -0.5, 0.0),
    (3.0, -3.0, 0.75),
    (0.0, 3.0, -1.5),
    (0.0, 0.0, 0.75),
    (-0.2, 0.5, 0.0),
)


def _attn_body(f0_ref, f1_ref, f2_ref, Wf1T, bf1, wf2, W5T, b5, out_o):
    f0, f1, f2 = f0_ref[:], f1_ref[:], f2_ref[:]
    hs = [t0 * f0 + t1 * f1 + t2 * f2 for (t0, t1, t2) in _THETA_W]
    ps = []
    for hk in hs:
        sk = jnp.tanh(jnp.dot(hk, Wf1T[:], preferred_element_type=jnp.float32)
                      + bf1[:])
        ps.append(jnp.sum(sk * wf2[:], axis=1, keepdims=True))
    m = ps[0]
    for pk in ps[1:]:
        m = jnp.maximum(m, pk)
    es = [jnp.exp(pk - m) for pk in ps]
    z = es[0]
    for ek in es[1:]:
        z = z + ek
    inv_z = 1.0 / z
    res = jnp.zeros((BN, H), jnp.float32)
    for ek, hk in zip(es, hs):
        res = res + (ek * inv_z) * hk
    out_o[:] = jnp.dot(res, W5T[:], preferred_element_type=jnp.float32) + b5[:]


def _attention(f0, f1, f2, Wf1T, bf1, wf2, W5T, b5):
    return pl.pallas_call(
        _attn_body,
        grid=(GRID,),
        in_specs=[pl.BlockSpec((BN, H), lambda i: (i, 0))] * 3
        + [_full(Wf1T.shape), _full(bf1.shape), _full(wf2.shape),
           _full(W5T.shape), _full(b5.shape)],
        out_specs=pl.BlockSpec((BN, H), lambda i: (i, 0)),
        out_shape=jax.ShapeDtypeStruct((NPAD, H), jnp.float32),
    )(f0, f1, f2, Wf1T, bf1, wf2, W5T, b5)


def _final_body(h0_ref, h1_ref, h2_ref, x0_ref, x1_ref, x2_ref, W6T, b6, out_o):
    hcat = _leaky(jnp.concatenate([h0_ref[:], h1_ref[:], h2_ref[:]], axis=1))
    full = jnp.concatenate([hcat, x0_ref[:], x1_ref[:], x2_ref[:]], axis=1)
    out_o[:] = jnp.dot(full, W6T[:], preferred_element_type=jnp.float32) + b6[:]


def _final(h0, h1, h2, x0, x1, x2, W6T, b6):
    return pl.pallas_call(
        _final_body,
        grid=(GRID,),
        in_specs=[pl.BlockSpec((BN, H), lambda i: (i, 0))] * 6
        + [_full(W6T.shape), _full(b6.shape)],
        out_specs=pl.BlockSpec((BN, 128), lambda i: (i, 0)),
        out_shape=jax.ShapeDtypeStruct((NPAD, 128), jnp.float32),
    )(h0, h1, h2, x0, x1, x2, W6T, b6)


def _pad_nodes(x):
    return jnp.pad(x, ((0, NPAD - N),) + ((0, 0),) * (x.ndim - 1))


def kernel(voc_features, sms_features, personal_feature,
           edge_index0, edge_index1, edge_index2,
           lstm_voc_Wih, lstm_voc_Whh, lstm_voc_bih, lstm_voc_bhh,
           lstm_sms_Wih, lstm_sms_Whh, lstm_sms_bih, lstm_sms_bhh,
           lin_voc_W, lin_voc_b, lin_sms_W, lin_sms_b, lin_per_W, lin_per_b,
           lin2_W, lin2_b, lin3_W, lin3_b, lin4_W, lin4_b,
           Wf1_0, bf1_0, Wf2_0, Wf1_1, bf1_1, Wf2_1, Wf1_2, bf1_2, Wf2_2,
           lin5_W0, lin5_b0, lin5_W1, lin5_b1, lin5_W2, lin5_b2,
           lin6_W, lin6_b):
    f32 = jnp.float32
    xvt = _pad_axis1(jnp.swapaxes(voc_features, 0, 1))
    xst = _pad_axis1(jnp.swapaxes(sms_features, 0, 1))
    per = _pad_nodes(personal_feature)
    srcs, dsts, dsts_flat = [], [], []
    for e in (edge_index0, edge_index1, edge_index2):
        srcs.append(e[0].reshape(NW, NCHUNK, CH))
        dsts.append(e[1].reshape(NW, NCHUNK, CH))
        dsts_flat.append(e[1].reshape(NW, 1, EW))
    zeros_h = jnp.zeros((NPAD, H), f32)
    zeros1 = jnp.zeros((NPAD,), f32)

    enc_w = [
        lstm_voc_Wih.T, lstm_voc_Whh.T,
        (lstm_voc_bih + lstm_voc_bhh).reshape(1, 4 * HH),
        lstm_sms_Wih.T, lstm_sms_Whh.T,
        (lstm_sms_bih + lstm_sms_bhh).reshape(1, 4 * HH),
        lin_voc_W.T, lin_voc_b.reshape(1, H),
        lin_sms_W.T, lin_sms_b.reshape(1, H),
        lin_per_W.T, lin_per_b.reshape(1, H),
        lin2_W.T, lin2_b.reshape(1, H),
        lin3_W.T, lin3_b.reshape(1, H),
        lin4_W.T, lin4_b.reshape(1, H),
    ]

    degp = _deg_partials(dsts_flat[0], dsts_flat[1], dsts_flat[2], zeros1)
    degp = degp.reshape(NW, 3, NPAD)

    x0, x1, x2, y00, y01, y02, dinv = _encoder(xvt, xst, per, degp, enc_w)
    xins = (x0, x1, x2)
    y0s = (y00, y01, y02)

    att_w = (
        (Wf1_0.T, bf1_0.reshape(1, H), Wf2_0, lin5_W0.T, lin5_b0.reshape(1, H)),
        (Wf1_1.T, bf1_1.reshape(1, H), Wf2_1, lin5_W1.T, lin5_b1.reshape(1, H)),
        (Wf1_2.T, bf1_2.reshape(1, H), Wf2_2, lin5_W2.T, lin5_b2.reshape(1, H)),
    )

    hs = []
    for r in range(3):
        dinv_r = dinv[r]
        p1 = _segment_partials(y0s[r], srcs[r], dsts[r], zeros_h)
        f1, y1 = _update(xins[r], p1, dinv_r)
        p2 = _segment_partials(y1, srcs[r], dsts[r], zeros_h)
        f2, _ = _update(f1, p2, dinv_r)
        hs.append(_attention(xins[r], f1, f2, *att_w[r]))

    W6T = jnp.zeros((6 * H, 128), f32).at[:, :C].set(lin6_W.T)
    b6 = jnp.zeros((1, 128), f32).at[0, :C].set(lin6_b)
    out = _final(hs[0], hs[1], hs[2], x0, x1, x2, W6T, b6)
    return out[:N, :C]


def _pad_axis1(x):
    return jnp.pad(x, ((0, 0), (0, NPAD - N), (0, 0)))


# speedup vs baseline: 10.8160x; 1.0554x over previous
import functools

import jax
import jax.numpy as jnp
from jax import lax
from jax.experimental import pallas as pl
from jax.experimental.pallas import tpu as pltpu, tpu_sc as plsc

N = 10000
T = 20
DV = 32
DS = 32
DP = 16
HH = 32
H = 64
C = 2
E = 640000

NPAD = 10240
NC = 2
NS = 16
NW = NC * NS
EW = E // NW
CH = 80
NCHUNK = EW // CH

BN = 2048
GRID = NPAD // BN

_RPT = NPAD // NS


def _sc_pass_body(y_hbm, src_hbm, dst_hbm, zeros_hbm, out_hbm,
                  src_v, dst_v, rows_a, rows_b, acc_sh, sem_a, sem_b):
    c = lax.axis_index("c")
    s = lax.axis_index("s")
    wid = s * NC + c
    pltpu.sync_copy(src_hbm.at[wid], src_v)
    pltpu.sync_copy(dst_hbm.at[wid], dst_v)
    pltpu.sync_copy(zeros_hbm.at[pl.ds(s * _RPT, _RPT)],
                    acc_sh.at[pl.ds(s * _RPT, _RPT)])
    plsc.subcore_barrier()

    pltpu.async_copy(y_hbm.at[src_v.at[0]], rows_a, sem_a)

    def body(jj, carry):
        j = 2 * jj
        pltpu.async_copy(y_hbm.at[src_v.at[j + 1]], rows_b, sem_b)
        pltpu.make_async_copy(y_hbm.at[src_v.at[j]], rows_a, sem_a).wait()
        pltpu.sync_copy(rows_a, acc_sh.at[dst_v.at[j]], add=True)

        @pl.when(j + 2 < NCHUNK)
        def _():
            pltpu.async_copy(y_hbm.at[src_v.at[j + 2]], rows_a, sem_a)

        pltpu.make_async_copy(y_hbm.at[src_v.at[j + 1]], rows_b, sem_b).wait()
        pltpu.sync_copy(rows_b, acc_sh.at[dst_v.at[j + 1]], add=True)
        return carry

    lax.fori_loop(0, NCHUNK // 2, body, 0)
    plsc.subcore_barrier()
    pltpu.sync_copy(acc_sh.at[pl.ds(s * _RPT, _RPT)],
                    out_hbm.at[c, pl.ds(s * _RPT, _RPT)])


def _sc_deg_body(dst0_hbm, dst1_hbm, dst2_hbm, zeros1_hbm, out_hbm,
                 idx_v, deg0_v, deg1_v, deg2_v):
    c = lax.axis_index("c")
    s = lax.axis_index("s")
    wid = s * NC + c
    degs = (deg0_v, deg1_v, deg2_v)
    for d_v in degs:
        pltpu.sync_copy(zeros1_hbm, d_v)
    ones16 = jnp.ones((16,), jnp.float32)
    for dst_hbm, d_v in zip((dst0_hbm, dst1_hbm, dst2_hbm), degs):
        pltpu.sync_copy(dst_hbm.at[wid, 0], idx_v)

        def body(k, carry, d_v=d_v):
            idx16 = idx_v[pl.ds(k * 16, 16)]
            plsc.addupdate_scatter(d_v, [idx16], ones16)
            return carry

        lax.fori_loop(0, EW // 16, body, 0)
    for r, d_v in enumerate(degs):
        pltpu.sync_copy(d_v, out_hbm.at[wid, 0, pl.ds(r * NPAD, NPAD)])


@functools.lru_cache(maxsize=None)
def _build_sc_kernels():
    mesh = plsc.VectorSubcoreMesh(core_axis_name="c", subcore_axis_name="s")
    params = pltpu.CompilerParams(use_tc_tiling_on_sc=False,
                                  needs_layout_passes=False)
    sc_pass = pl.kernel(
        _sc_pass_body,
        out_type=jax.ShapeDtypeStruct((NC, NPAD, H), jnp.float32),
        mesh=mesh,
        compiler_params=params,
        scratch_types=[
            pltpu.VMEM((NCHUNK, CH), jnp.int32),
            pltpu.VMEM((NCHUNK, CH), jnp.int32),
            pltpu.VMEM((CH, H), jnp.float32),
            pltpu.VMEM((CH, H), jnp.float32),
            pltpu.VMEM_SHARED((NPAD, H), jnp.float32),
            pltpu.SemaphoreType.DMA,
            pltpu.SemaphoreType.DMA,
        ],
    )
    sc_deg = pl.kernel(
        _sc_deg_body,
        out_type=jax.ShapeDtypeStruct((NW, 1, 3 * NPAD), jnp.float32),
        mesh=mesh,
        compiler_params=params,
        scratch_types=[
            pltpu.VMEM((EW,), jnp.int32),
            pltpu.VMEM((NPAD,), jnp.float32),
            pltpu.VMEM((NPAD,), jnp.float32),
            pltpu.VMEM((NPAD,), jnp.float32),
        ],
    )
    return sc_pass, sc_deg


def _segment_partials(y, src_w, dst_w, zeros_h):
    return _build_sc_kernels()[0](y, src_w, dst_w, zeros_h)


def _deg_partials(dst0_w, dst1_w, dst2_w, zeros1):
    return _build_sc_kernels()[1](dst0_w, dst1_w, dst2_w, zeros1)


def _full(shape):
    return pl.BlockSpec(shape, lambda i: (0,) * len(shape))


def _lstm_flat(x_ref, WiT, WhT, b):
    h = jnp.zeros((BN, HH), jnp.float32)
    c = jnp.zeros((BN, HH), jnp.float32)
    for t in range(T):
        xt = x_ref[:, t * DV:(t + 1) * DV]
        g = jnp.dot(xt, WiT, preferred_element_type=jnp.float32)
        g = g + jnp.dot(h, WhT, preferred_element_type=jnp.float32) + b
        i = jax.nn.sigmoid(g[:, 0 * HH:1 * HH])
        f = jax.nn.sigmoid(g[:, 1 * HH:2 * HH])
        gg = jnp.tanh(g[:, 2 * HH:3 * HH])
        o = jax.nn.sigmoid(g[:, 3 * HH:4 * HH])
        c = f * c + i * gg
        h = o * jnp.tanh(c)
    return h


def _leaky(x):
    return jnp.where(x >= 0, x, 0.01 * x)


def _encoder_body(xvf_ref, xsf_ref, per_ref, degp_ref,
                  WivT, WhvT, bv, WisT, WhsT, bs,
                  LvT, lbv, LsT, lbs, LpT, lbp,
                  W2T, b2, W3T, b3, W4T, b4,
                  x0_o, x1_o, x2_o, y0_o, y1_o, y2_o, dinv_o):
    hv = _lstm_flat(xvf_ref, WivT[:], WhvT[:], bv[:])
    hs = _lstm_flat(xsf_ref, WisT[:], WhsT[:], bs[:])
    xv = jnp.dot(hv, LvT[:], preferred_element_type=jnp.float32) + lbv[:]
    xs = jnp.dot(hs, LsT[:], preferred_element_type=jnp.float32) + lbs[:]
    xp = jnp.dot(per_ref[:], LpT[:], preferred_element_type=jnp.float32) + lbp[:]
    xv2 = _leaky(jnp.concatenate([xv, xp], axis=1))
    xs2 = _leaky(jnp.concatenate([xs, xp], axis=1))
    xv3 = _leaky(jnp.dot(xv2, W2T[:], preferred_element_type=jnp.float32) + b2[:])
    xs3 = _leaky(jnp.dot(xs2, W3T[:], preferred_element_type=jnp.float32) + b3[:])
    xvs = jnp.concatenate([xv2, xs2], axis=1)
    xvs3 = _leaky(jnp.dot(xvs, W4T[:], preferred_element_type=jnp.float32) + b4[:])
    x0_o[:] = xv3
    x1_o[:] = xs3
    x2_o[:] = xvs3
    xins = (xv3, xs3, xvs3)
    youts = (y0_o, y1_o, y2_o)
    deg3 = jnp.sum(degp_ref[:], axis=0)
    for r in range(3):
        dinv = lax.rsqrt(jnp.maximum(deg3[r], 1.0)).reshape(BN, 1)
        dinv_o[r] = dinv
        youts[r][:] = xins[r] * dinv


def _encoder(xvf, xsf, per, degp, weights):
    outs = (
        [jax.ShapeDtypeStruct((NPAD, H), jnp.float32)] * 6
        + [jax.ShapeDtypeStruct((3, NPAD, 1), jnp.float32)]
    )
    out_specs = (
        [pl.BlockSpec((BN, H), lambda i: (i, 0))] * 6
        + [pl.BlockSpec((3, BN, 1), lambda i: (0, i, 0))]
    )
    in_specs = [
        pl.BlockSpec((BN, T * DV), lambda i: (i, 0)),
        pl.BlockSpec((BN, T * DS), lambda i: (i, 0)),
        pl.BlockSpec((BN, DP), lambda i: (i, 0)),
        pl.BlockSpec((NW, 3, BN), lambda i: (0, 0, i)),
    ] + [_full(w.shape) for w in weights]
    return pl.pallas_call(
        _encoder_body,
        grid=(GRID,),
        in_specs=in_specs,
        out_specs=out_specs,
        out_shape=outs,
    )(xvf, xsf, per, degp, *weights)


def _update_body(x_ref, p_ref, dinv_ref, f_o, y_o):
    agg = p_ref[0] + p_ref[1]
    dinv = dinv_ref[:]
    f = x_ref[:] - agg * dinv
    f_o[:] = f
    y_o[:] = f * dinv


def _update(x, partials, dinv_r):
    return pl.pallas_call(
        _update_body,
        grid=(GRID,),
        in_specs=[
            pl.BlockSpec((BN, H), lambda i: (i, 0)),
            pl.BlockSpec((NC, BN, H), lambda i: (0, i, 0)),
            pl.BlockSpec((BN, 1), lambda i: (i, 0)),
        ],
        out_specs=[pl.BlockSpec((BN, H), lambda i: (i, 0))] * 2,
        out_shape=[jax.ShapeDtypeStruct((NPAD, H), jnp.float32)] * 2,
    )(x, partials, dinv_r)


_THETA_W = (
    (0.8, -0.5, 0.0),
    (3.0, -3.0, 0.75),
    (0.0, 3.0, -1.5),
    (0.0, 0.0, 0.75),
    (-0.2, 0.5, 0.0),
)


def _attn_body(f0_ref, f1_ref, f2_ref, Wf1T, bf1, wf2, W5T, b5, out_o):
    f0, f1, f2 = f0_ref[:], f1_ref[:], f2_ref[:]
    hs = [t0 * f0 + t1 * f1 + t2 * f2 for (t0, t1, t2) in _THETA_W]
    ps = []
    for hk in hs:
        sk = jnp.tanh(jnp.dot(hk, Wf1T[:], preferred_element_type=jnp.float32)
                      + bf1[:])
        ps.append(jnp.sum(sk * wf2[:], axis=1, keepdims=True))
    m = ps[0]
    for pk in ps[1:]:
        m = jnp.maximum(m, pk)
    es = [jnp.exp(pk - m) for pk in ps]
    z = es[0]
    for ek in es[1:]:
        z = z + ek
    inv_z = 1.0 / z
    res = jnp.zeros((BN, H), jnp.float32)
    for ek, hk in zip(es, hs):
        res = res + (ek * inv_z) * hk
    out_o[:] = jnp.dot(res, W5T[:], preferred_element_type=jnp.float32) + b5[:]


def _attention(f0, f1, f2, Wf1T, bf1, wf2, W5T, b5):
    return pl.pallas_call(
        _attn_body,
        grid=(GRID,),
        in_specs=[pl.BlockSpec((BN, H), lambda i: (i, 0))] * 3
        + [_full(Wf1T.shape), _full(bf1.shape), _full(wf2.shape),
           _full(W5T.shape), _full(b5.shape)],
        out_specs=pl.BlockSpec((BN, H), lambda i: (i, 0)),
        out_shape=jax.ShapeDtypeStruct((NPAD, H), jnp.float32),
    )(f0, f1, f2, Wf1T, bf1, wf2, W5T, b5)


def _final_body(h0_ref, h1_ref, h2_ref, x0_ref, x1_ref, x2_ref, W6T, b6, out_o):
    hcat = _leaky(jnp.concatenate([h0_ref[:], h1_ref[:], h2_ref[:]], axis=1))
    full = jnp.concatenate([hcat, x0_ref[:], x1_ref[:], x2_ref[:]], axis=1)
    out_o[:] = jnp.dot(full, W6T[:], preferred_element_type=jnp.float32) + b6[:]


def _final(h0, h1, h2, x0, x1, x2, W6T, b6):
    return pl.pallas_call(
        _final_body,
        grid=(GRID,),
        in_specs=[pl.BlockSpec((BN, H), lambda i: (i, 0))] * 6
        + [_full(W6T.shape), _full(b6.shape)],
        out_specs=pl.BlockSpec((BN, 128), lambda i: (i, 0)),
        out_shape=jax.ShapeDtypeStruct((NPAD, 128), jnp.float32),
    )(h0, h1, h2, x0, x1, x2, W6T, b6)


def kernel(voc_features, sms_features, personal_feature,
           edge_index0, edge_index1, edge_index2,
           lstm_voc_Wih, lstm_voc_Whh, lstm_voc_bih, lstm_voc_bhh,
           lstm_sms_Wih, lstm_sms_Whh, lstm_sms_bih, lstm_sms_bhh,
           lin_voc_W, lin_voc_b, lin_sms_W, lin_sms_b, lin_per_W, lin_per_b,
           lin2_W, lin2_b, lin3_W, lin3_b, lin4_W, lin4_b,
           Wf1_0, bf1_0, Wf2_0, Wf1_1, bf1_1, Wf2_1, Wf1_2, bf1_2, Wf2_2,
           lin5_W0, lin5_b0, lin5_W1, lin5_b1, lin5_W2, lin5_b2,
           lin6_W, lin6_b):
    f32 = jnp.float32
    xvf = _pad_rows(voc_features.reshape(N, T * DV))
    xsf = _pad_rows(sms_features.reshape(N, T * DS))
    per = _pad_rows(personal_feature)
    srcs, dsts, dsts_flat = [], [], []
    for e in (edge_index0, edge_index1, edge_index2):
        srcs.append(e[0].reshape(NW, NCHUNK, CH))
        dsts.append(e[1].reshape(NW, NCHUNK, CH))
        dsts_flat.append(e[1].reshape(NW, 1, EW))
    zeros_h = jnp.zeros((NPAD, H), f32)
    zeros1 = jnp.zeros((NPAD,), f32)

    enc_w = [
        lstm_voc_Wih.T, lstm_voc_Whh.T,
        (lstm_voc_bih + lstm_voc_bhh).reshape(1, 4 * HH),
        lstm_sms_Wih.T, lstm_sms_Whh.T,
        (lstm_sms_bih + lstm_sms_bhh).reshape(1, 4 * HH),
        lin_voc_W.T, lin_voc_b.reshape(1, H),
        lin_sms_W.T, lin_sms_b.reshape(1, H),
        lin_per_W.T, lin_per_b.reshape(1, H),
        lin2_W.T, lin2_b.reshape(1, H),
        lin3_W.T, lin3_b.reshape(1, H),
        lin4_W.T, lin4_b.reshape(1, H),
    ]

    degp = _deg_partials(dsts_flat[0], dsts_flat[1], dsts_flat[2], zeros1)
    degp = degp.reshape(NW, 3, NPAD)

    x0, x1, x2, y00, y01, y02, dinv = _encoder(xvf, xsf, per, degp, enc_w)
    xins = (x0, x1, x2)
    y0s = (y00, y01, y02)

    att_w = (
        (Wf1_0.T, bf1_0.reshape(1, H), Wf2_0, lin5_W0.T, lin5_b0.reshape(1, H)),
        (Wf1_1.T, bf1_1.reshape(1, H), Wf2_1, lin5_W1.T, lin5_b1.reshape(1, H)),
        (Wf1_2.T, bf1_2.reshape(1, H), Wf2_2, lin5_W2.T, lin5_b2.reshape(1, H)),
    )

    hs = []
    for r in range(3):
        dinv_r = dinv[r]
        p1 = _segment_partials(y0s[r], srcs[r], dsts[r], zeros_h)
        f1, y1 = _update(xins[r], p1, dinv_r)
        p2 = _segment_partials(y1, srcs[r], dsts[r], zeros_h)
        f2, _ = _update(f1, p2, dinv_r)
        hs.append(_attention(xins[r], f1, f2, *att_w[r]))

    W6T = jnp.zeros((6 * H, 128), f32).at[:, :C].set(lin6_W.T)
    b6 = jnp.zeros((1, 128), f32).at[0, :C].set(lin6_b)
    out = _final(hs[0], hs[1], hs[2], x0, x1, x2, W6T, b6)
    return out[:N, :C]


def _pad_rows(x):
    return jnp.pad(x, ((0, NPAD - N), (0, 0)))


# speedup vs baseline: 11.1464x; 1.0305x over previous
import functools

import jax
import jax.numpy as jnp
from jax import lax
from jax.experimental import pallas as pl
from jax.experimental.pallas import tpu as pltpu, tpu_sc as plsc

N = 10000
T = 20
DV = 32
DS = 32
DP = 16
HH = 32
H = 64
C = 2
E = 640000

NPAD = 10240
NC = 2
NS = 16
NW = NC * NS
EW = E // NW
CH = 80
NCHUNK = EW // CH

BN = 2048
GRID = NPAD // BN

_RPT = NPAD // NS


def _sc_pass_gen(nrel):
    def body(*refs):
        ys = refs[0:nrel]
        srcs_h = refs[nrel:2 * nrel]
        dsts_h = refs[2 * nrel:3 * nrel]
        zeros_hbm = refs[3 * nrel]
        out_hbm = refs[3 * nrel + 1]
        src_v, dst_v, rows_a, rows_b = refs[3 * nrel + 2:3 * nrel + 6]
        accs = refs[3 * nrel + 6:3 * nrel + 6 + nrel]
        sem_a, sem_b = refs[3 * nrel + 6 + nrel:]
        c = lax.axis_index("c")
        s = lax.axis_index("s")
        wid = s * NC + c
        for acc_sh in accs:
            pltpu.sync_copy(zeros_hbm.at[pl.ds(s * _RPT, _RPT)],
                            acc_sh.at[pl.ds(s * _RPT, _RPT)])
        plsc.subcore_barrier()
        for y_hbm, src_hbm, dst_hbm, acc_sh in zip(ys, srcs_h, dsts_h, accs):
            pltpu.sync_copy(src_hbm.at[wid], src_v)
            pltpu.sync_copy(dst_hbm.at[wid], dst_v)
            pltpu.async_copy(y_hbm.at[src_v.at[0]], rows_a, sem_a)

            def lbody(jj, carry, y_hbm=y_hbm, acc_sh=acc_sh):
                j = 2 * jj
                pltpu.async_copy(y_hbm.at[src_v.at[j + 1]], rows_b, sem_b)
                pltpu.make_async_copy(y_hbm.at[src_v.at[j]], rows_a,
                                      sem_a).wait()
                pltpu.sync_copy(rows_a, acc_sh.at[dst_v.at[j]], add=True)

                @pl.when(j + 2 < NCHUNK)
                def _():
                    pltpu.async_copy(y_hbm.at[src_v.at[j + 2]], rows_a, sem_a)

                pltpu.make_async_copy(y_hbm.at[src_v.at[j + 1]], rows_b,
                                      sem_b).wait()
                pltpu.sync_copy(rows_b, acc_sh.at[dst_v.at[j + 1]], add=True)
                return carry

            lax.fori_loop(0, NCHUNK // 2, lbody, 0)
        plsc.subcore_barrier()
        for r, acc_sh in enumerate(accs):
            pltpu.sync_copy(acc_sh.at[pl.ds(s * _RPT, _RPT)],
                            out_hbm.at[c, r, pl.ds(s * _RPT, _RPT)])
    return body


def _sc_deg_body(dst0_hbm, dst1_hbm, dst2_hbm, zeros1_hbm, out_hbm,
                 idx_v, deg0_v, deg1_v, deg2_v):
    c = lax.axis_index("c")
    s = lax.axis_index("s")
    wid = s * NC + c
    degs = (deg0_v, deg1_v, deg2_v)
    for d_v in degs:
        pltpu.sync_copy(zeros1_hbm, d_v)
    ones16 = jnp.ones((16,), jnp.float32)
    for dst_hbm, d_v in zip((dst0_hbm, dst1_hbm, dst2_hbm), degs):
        pltpu.sync_copy(dst_hbm.at[wid, 0], idx_v)

        def body(k, carry, d_v=d_v):
            idx16 = idx_v[pl.ds(k * 16, 16)]
            plsc.addupdate_scatter(d_v, [idx16], ones16)
            return carry

        lax.fori_loop(0, EW // 16, body, 0)
    for r, d_v in enumerate(degs):
        pltpu.sync_copy(d_v, out_hbm.at[wid, 0, pl.ds(r * NPAD, NPAD)])


@functools.lru_cache(maxsize=None)
def _build_sc_kernels():
    mesh = plsc.VectorSubcoreMesh(core_axis_name="c", subcore_axis_name="s")
    params = pltpu.CompilerParams(use_tc_tiling_on_sc=False,
                                  needs_layout_passes=False)

    def make_pass(nrel):
        return pl.kernel(
            _sc_pass_gen(nrel),
            out_type=jax.ShapeDtypeStruct((NC, nrel, NPAD, H), jnp.float32),
            mesh=mesh,
            compiler_params=params,
            scratch_types=[
                pltpu.VMEM((NCHUNK, CH), jnp.int32),
                pltpu.VMEM((NCHUNK, CH), jnp.int32),
                pltpu.VMEM((CH, H), jnp.float32),
                pltpu.VMEM((CH, H), jnp.float32),
            ] + [pltpu.VMEM_SHARED((NPAD, H), jnp.float32)] * nrel + [
                pltpu.SemaphoreType.DMA,
                pltpu.SemaphoreType.DMA,
            ],
        )

    sc_pass1 = make_pass(1)
    sc_deg = pl.kernel(
        _sc_deg_body,
        out_type=jax.ShapeDtypeStruct((NW, 1, 3 * NPAD), jnp.float32),
        mesh=mesh,
        compiler_params=params,
        scratch_types=[
            pltpu.VMEM((EW,), jnp.int32),
            pltpu.VMEM((NPAD,), jnp.float32),
            pltpu.VMEM((NPAD,), jnp.float32),
            pltpu.VMEM((NPAD,), jnp.float32),
        ],
    )
    return sc_pass1, sc_deg


def _segment_partials(y, src_w, dst_w, zeros_h):
    return _build_sc_kernels()[0](y, src_w, dst_w, zeros_h)


def _deg_partials(dst0_w, dst1_w, dst2_w, zeros1):
    return _build_sc_kernels()[1](dst0_w, dst1_w, dst2_w, zeros1)


def _full(shape):
    return pl.BlockSpec(shape, lambda i: (0,) * len(shape))


def _lstm_flat(x_ref, WiT, WhT, b):
    h = jnp.zeros((BN, HH), jnp.float32)
    c = jnp.zeros((BN, HH), jnp.float32)
    for t in range(T):
        xt = x_ref[:, t * DV:(t + 1) * DV]
        g = jnp.dot(xt, WiT, preferred_element_type=jnp.float32)
        g = g + jnp.dot(h, WhT, preferred_element_type=jnp.float32) + b
        i = jax.nn.sigmoid(g[:, 0 * HH:1 * HH])
        f = jax.nn.sigmoid(g[:, 1 * HH:2 * HH])
        gg = jnp.tanh(g[:, 2 * HH:3 * HH])
        o = jax.nn.sigmoid(g[:, 3 * HH:4 * HH])
        c = f * c + i * gg
        h = o * jnp.tanh(c)
    return h


def _leaky(x):
    return jnp.where(x >= 0, x, 0.01 * x)


def _encoder_body(xvf_ref, xsf_ref, per_ref,
                  WivT, WhvT, bv, WisT, WhsT, bs,
                  LvT, lbv, LsT, lbs, LpT, lbp,
                  W2T, b2, W3T, b3, W4T, b4,
                  x0_o, x1_o, x2_o):
    hv = _lstm_flat(xvf_ref, WivT[:], WhvT[:], bv[:])
    hs = _lstm_flat(xsf_ref, WisT[:], WhsT[:], bs[:])
    xv = jnp.dot(hv, LvT[:], preferred_element_type=jnp.float32) + lbv[:]
    xs = jnp.dot(hs, LsT[:], preferred_element_type=jnp.float32) + lbs[:]
    xp = jnp.dot(per_ref[:], LpT[:], preferred_element_type=jnp.float32) + lbp[:]
    xv2 = _leaky(jnp.concatenate([xv, xp], axis=1))
    xs2 = _leaky(jnp.concatenate([xs, xp], axis=1))
    x0_o[:] = _leaky(jnp.dot(xv2, W2T[:], preferred_element_type=jnp.float32)
                     + b2[:])
    x1_o[:] = _leaky(jnp.dot(xs2, W3T[:], preferred_element_type=jnp.float32)
                     + b3[:])
    xvs = jnp.concatenate([xv2, xs2], axis=1)
    x2_o[:] = _leaky(jnp.dot(xvs, W4T[:], preferred_element_type=jnp.float32)
                     + b4[:])


def _encoder(xvf, xsf, per, weights):
    in_specs = [
        pl.BlockSpec((BN, T * DV), lambda i: (i, 0)),
        pl.BlockSpec((BN, T * DS), lambda i: (i, 0)),
        pl.BlockSpec((BN, DP), lambda i: (i, 0)),
    ] + [_full(w.shape) for w in weights]
    return pl.pallas_call(
        _encoder_body,
        grid=(GRID,),
        in_specs=in_specs,
        out_specs=[pl.BlockSpec((BN, H), lambda i: (i, 0))] * 3,
        out_shape=[jax.ShapeDtypeStruct((NPAD, H), jnp.float32)] * 3,
    )(xvf, xsf, per, *weights)


def _prep_body(degp_ref, x0_ref, x1_ref, x2_ref,
               dinv_o, y0_o, y1_o, y2_o):
    deg3 = jnp.sum(degp_ref[:], axis=0)
    xins = (x0_ref, x1_ref, x2_ref)
    youts = (y0_o, y1_o, y2_o)
    for r in range(3):
        dinv = lax.rsqrt(jnp.maximum(deg3[r], 1.0)).reshape(BN, 1)
        dinv_o[r] = dinv
        youts[r][:] = xins[r][:] * dinv


def _prep(degp, x0, x1, x2):
    outs = ([jax.ShapeDtypeStruct((3, NPAD, 1), jnp.float32)]
            + [jax.ShapeDtypeStruct((NPAD, H), jnp.float32)] * 3)
    out_specs = ([pl.BlockSpec((3, BN, 1), lambda i: (0, i, 0))]
                 + [pl.BlockSpec((BN, H), lambda i: (i, 0))] * 3)
    return pl.pallas_call(
        _prep_body,
        grid=(GRID,),
        in_specs=[pl.BlockSpec((NW, 3, BN), lambda i: (0, 0, i))]
        + [pl.BlockSpec((BN, H), lambda i: (i, 0))] * 3,
        out_specs=out_specs,
        out_shape=outs,
    )(degp, x0, x1, x2)


def _update_body(x_ref, p_ref, dinv_ref, f_o, y_o):
    agg = p_ref[0, 0] + p_ref[1, 0]
    dinv = dinv_ref[0]
    f = x_ref[:] - agg * dinv
    f_o[:] = f
    y_o[:] = f * dinv


def _update(x, p, dinv_r):
    return pl.pallas_call(
        _update_body,
        grid=(GRID,),
        in_specs=[
            pl.BlockSpec((BN, H), lambda i: (i, 0)),
            pl.BlockSpec((NC, 1, BN, H), lambda i: (0, 0, i, 0)),
            pl.BlockSpec((1, BN, 1), lambda i: (0, i, 0)),
        ],
        out_specs=[pl.BlockSpec((BN, H), lambda i: (i, 0))] * 2,
        out_shape=[jax.ShapeDtypeStruct((NPAD, H), jnp.float32)] * 2,
    )(x, p, dinv_r)


_THETA_W = (
    (0.8, -0.5, 0.0),
    (3.0, -3.0, 0.75),
    (0.0, 3.0, -1.5),
    (0.0, 0.0, 0.75),
    (-0.2, 0.5, 0.0),
)


def _attn_one(f0, f1, f2, Wf1T, bf1, wf2, W5T, b5):
    hs = [t0 * f0 + t1 * f1 + t2 * f2 for (t0, t1, t2) in _THETA_W]
    ps = []
    for hk in hs:
        sk = jnp.tanh(jnp.dot(hk, Wf1T, preferred_element_type=jnp.float32)
                      + bf1)
        ps.append(jnp.sum(sk * wf2, axis=1, keepdims=True))
    m = ps[0]
    for pk in ps[1:]:
        m = jnp.maximum(m, pk)
    es = [jnp.exp(pk - m) for pk in ps]
    z = es[0]
    for ek in es[1:]:
        z = z + ek
    inv_z = 1.0 / z
    res = jnp.zeros(f0.shape, jnp.float32)
    for ek, hk in zip(es, hs):
        res = res + (ek * inv_z) * hk
    return jnp.dot(res, W5T, preferred_element_type=jnp.float32) + b5


def _tail_body(x0_ref, x1_ref, x2_ref, f10_ref, f11_ref, f12_ref,
               p0_ref, p1_ref, p2_ref, dinv_ref,
               Wf1T0, bf10, wf20, W5T0, b50,
               Wf1T1, bf11, wf21, W5T1, b51,
               Wf1T2, bf12, wf22, W5T2, b52,
               W6T, b6, out_o):
    xins = (x0_ref, x1_ref, x2_ref)
    f1s = (f10_ref, f11_ref, f12_ref)
    attw = ((Wf1T0, bf10, wf20, W5T0, b50),
            (Wf1T1, bf11, wf21, W5T1, b51),
            (Wf1T2, bf12, wf22, W5T2, b52))
    hs = []
    prefs = (p0_ref, p1_ref, p2_ref)
    for r in range(3):
        agg = prefs[r][0, 0] + prefs[r][1, 0]
        f2 = f1s[r][:] - agg * dinv_ref[r]
        Wf1T, bf1, wf2, W5T, b5 = attw[r]
        hs.append(_attn_one(xins[r][:], f1s[r][:], f2,
                            Wf1T[:], bf1[:], wf2[:], W5T[:], b5[:]))
    hcat = _leaky(jnp.concatenate(hs, axis=1))
    full = jnp.concatenate([hcat, xins[0][:], xins[1][:], xins[2][:]], axis=1)
    out_o[:] = jnp.dot(full, W6T[:], preferred_element_type=jnp.float32) + b6[:]


def _tail(x0, x1, x2, f10, f11, f12, p20, p21, p22, dinv, att_w, W6T, b6):
    flat_w = [w for ws in att_w for w in ws] + [W6T, b6]
    return pl.pallas_call(
        _tail_body,
        grid=(GRID,),
        in_specs=[pl.BlockSpec((BN, H), lambda i: (i, 0))] * 6
        + [pl.BlockSpec((NC, 1, BN, H), lambda i: (0, 0, i, 0))] * 3
        + [pl.BlockSpec((3, BN, 1), lambda i: (0, i, 0))]
        + [_full(w.shape) for w in flat_w],
        out_specs=pl.BlockSpec((BN, 128), lambda i: (i, 0)),
        out_shape=jax.ShapeDtypeStruct((NPAD, 128), jnp.float32),
    )(x0, x1, x2, f10, f11, f12, p20, p21, p22, dinv, *flat_w)


def kernel(voc_features, sms_features, personal_feature,
           edge_index0, edge_index1, edge_index2,
           lstm_voc_Wih, lstm_voc_Whh, lstm_voc_bih, lstm_voc_bhh,
           lstm_sms_Wih, lstm_sms_Whh, lstm_sms_bih, lstm_sms_bhh,
           lin_voc_W, lin_voc_b, lin_sms_W, lin_sms_b, lin_per_W, lin_per_b,
           lin2_W, lin2_b, lin3_W, lin3_b, lin4_W, lin4_b,
           Wf1_0, bf1_0, Wf2_0, Wf1_1, bf1_1, Wf2_1, Wf1_2, bf1_2, Wf2_2,
           lin5_W0, lin5_b0, lin5_W1, lin5_b1, lin5_W2, lin5_b2,
           lin6_W, lin6_b):
    f32 = jnp.float32
    xvf = _pad_rows(voc_features.reshape(N, T * DV))
    xsf = _pad_rows(sms_features.reshape(N, T * DS))
    per = _pad_rows(personal_feature)
    srcs, dsts, dsts_flat = [], [], []
    for e in (edge_index0, edge_index1, edge_index2):
        srcs.append(e[0].reshape(NW, NCHUNK, CH))
        dsts.append(e[1].reshape(NW, NCHUNK, CH))
        dsts_flat.append(e[1].reshape(NW, 1, EW))
    zeros_h = jnp.zeros((NPAD, H), f32)
    zeros1 = jnp.zeros((NPAD,), f32)

    enc_w = [
        lstm_voc_Wih.T, lstm_voc_Whh.T,
        (lstm_voc_bih + lstm_voc_bhh).reshape(1, 4 * HH),
        lstm_sms_Wih.T, lstm_sms_Whh.T,
        (lstm_sms_bih + lstm_sms_bhh).reshape(1, 4 * HH),
        lin_voc_W.T, lin_voc_b.reshape(1, H),
        lin_sms_W.T, lin_sms_b.reshape(1, H),
        lin_per_W.T, lin_per_b.reshape(1, H),
        lin2_W.T, lin2_b.reshape(1, H),
        lin3_W.T, lin3_b.reshape(1, H),
        lin4_W.T, lin4_b.reshape(1, H),
    ]

    degp = _deg_partials(dsts_flat[0], dsts_flat[1], dsts_flat[2], zeros1)
    degp = degp.reshape(NW, 3, NPAD)
    x0, x1, x2 = _encoder(xvf, xsf, per, enc_w)
    dinv, y00, y01, y02 = _prep(degp, x0, x1, x2)

    att_w = (
        (Wf1_0.T, bf1_0.reshape(1, H), Wf2_0, lin5_W0.T, lin5_b0.reshape(1, H)),
        (Wf1_1.T, bf1_1.reshape(1, H), Wf2_1, lin5_W1.T, lin5_b1.reshape(1, H)),
        (Wf1_2.T, bf1_2.reshape(1, H), Wf2_2, lin5_W2.T, lin5_b2.reshape(1, H)),
    )

    xs = (x0, x1, x2)
    y0s = (y00, y01, y02)
    f1s, p2s = [], []
    for r in range(3):
        p1r = _segment_partials(y0s[r], srcs[r], dsts[r], zeros_h)
        fr, zr = _update(xs[r], p1r, dinv[r:r + 1])
        f1s.append(fr)
        p2s.append(_segment_partials(zr, srcs[r], dsts[r], zeros_h))

    W6T = jnp.zeros((6 * H, 128), f32).at[:, :C].set(lin6_W.T)
    b6 = jnp.zeros((1, 128), f32).at[0, :C].set(lin6_b)
    out = _tail(x0, x1, x2, f1s[0], f1s[1], f1s[2],
                p2s[0], p2s[1], p2s[2], dinv, att_w, W6T, b6)
    return out[:N, :C]


def _pad_rows(x):
    return jnp.pad(x, ((0, NPAD - N), (0, 0)))


# speedup vs baseline: 11.2063x; 1.0054x over previous
import functools

import jax
import jax.numpy as jnp
from jax import lax
from jax.experimental import pallas as pl
from jax.experimental.pallas import tpu as pltpu, tpu_sc as plsc

N = 10000
T = 20
DV = 32
DS = 32
DP = 16
HH = 32
H = 64
C = 2
E = 640000

NPAD = 10240
NC = 2
NS = 16
NW = NC * NS
EW = E // NW
CH = 80
NCHUNK = EW // CH

BN = 2048
GRID = NPAD // BN

_RPT = NPAD // NS


def _sc_pass_gen(nrel):
    def body(*refs):
        ys = refs[0:nrel]
        srcs_h = refs[nrel:2 * nrel]
        dsts_h = refs[2 * nrel:3 * nrel]
        zeros_hbm = refs[3 * nrel]
        out_hbm = refs[3 * nrel + 1]
        src_v, dst_v, rows_a, rows_b = refs[3 * nrel + 2:3 * nrel + 6]
        accs = refs[3 * nrel + 6:3 * nrel + 6 + nrel]
        sem_a, sem_b = refs[3 * nrel + 6 + nrel:]
        c = lax.axis_index("c")
        s = lax.axis_index("s")
        wid = s * NC + c
        for acc_sh in accs:
            pltpu.sync_copy(zeros_hbm.at[pl.ds(s * _RPT, _RPT)],
                            acc_sh.at[pl.ds(s * _RPT, _RPT)])
        plsc.subcore_barrier()
        for y_hbm, src_hbm, dst_hbm, acc_sh in zip(ys, srcs_h, dsts_h, accs):
            pltpu.sync_copy(src_hbm.at[wid], src_v)
            pltpu.sync_copy(dst_hbm.at[wid], dst_v)
            pltpu.async_copy(y_hbm.at[src_v.at[0]], rows_a, sem_a)

            def lbody(jj, carry, y_hbm=y_hbm, acc_sh=acc_sh):
                j = 2 * jj
                pltpu.async_copy(y_hbm.at[src_v.at[j + 1]], rows_b, sem_b)
                pltpu.make_async_copy(y_hbm.at[src_v.at[j]], rows_a,
                                      sem_a).wait()
                pltpu.sync_copy(rows_a, acc_sh.at[dst_v.at[j]], add=True)

                @pl.when(j + 2 < NCHUNK)
                def _():
                    pltpu.async_copy(y_hbm.at[src_v.at[j + 2]], rows_a, sem_a)

                pltpu.make_async_copy(y_hbm.at[src_v.at[j + 1]], rows_b,
                                      sem_b).wait()
                pltpu.sync_copy(rows_b, acc_sh.at[dst_v.at[j + 1]], add=True)
                return carry

            lax.fori_loop(0, NCHUNK // 2, lbody, 0)
        plsc.subcore_barrier()
        for r, acc_sh in enumerate(accs):
            pltpu.sync_copy(acc_sh.at[pl.ds(s * _RPT, _RPT)],
                            out_hbm.at[c, r, pl.ds(s * _RPT, _RPT)])
    return body


def _sc_deg_body(dst0_hbm, dst1_hbm, dst2_hbm, zeros1_hbm, out_hbm,
                 idx_v, deg0_v, deg1_v, deg2_v):
    c = lax.axis_index("c")
    s = lax.axis_index("s")
    wid = s * NC + c
    degs = (deg0_v, deg1_v, deg2_v)
    for d_v in degs:
        pltpu.sync_copy(zeros1_hbm, d_v)
    ones16 = jnp.ones((16,), jnp.float32)
    for dst_hbm, d_v in zip((dst0_hbm, dst1_hbm, dst2_hbm), degs):
        pltpu.sync_copy(dst_hbm.at[wid, 0], idx_v)

        def body(k, carry, d_v=d_v):
            for m in range(8):
                idx16 = idx_v[pl.ds(k * 128 + m * 16, 16)]
                plsc.addupdate_scatter(d_v, [idx16], ones16)
            return carry

        lax.fori_loop(0, EW // 128, body, 0)
    for r, d_v in enumerate(degs):
        pltpu.sync_copy(d_v, out_hbm.at[wid, 0, pl.ds(r * NPAD, NPAD)])


@functools.lru_cache(maxsize=None)
def _build_sc_kernels():
    mesh = plsc.VectorSubcoreMesh(core_axis_name="c", subcore_axis_name="s")
    params = pltpu.CompilerParams(use_tc_tiling_on_sc=False,
                                  needs_layout_passes=False)

    def make_pass(nrel):
        return pl.kernel(
            _sc_pass_gen(nrel),
            out_type=jax.ShapeDtypeStruct((NC, nrel, NPAD, H), jnp.float32),
            mesh=mesh,
            compiler_params=params,
            scratch_types=[
                pltpu.VMEM((NCHUNK, CH), jnp.int32),
                pltpu.VMEM((NCHUNK, CH), jnp.int32),
                pltpu.VMEM((CH, H), jnp.float32),
                pltpu.VMEM((CH, H), jnp.float32),
            ] + [pltpu.VMEM_SHARED((NPAD, H), jnp.float32)] * nrel + [
                pltpu.SemaphoreType.DMA,
                pltpu.SemaphoreType.DMA,
            ],
        )

    sc_pass1 = make_pass(1)
    sc_deg = pl.kernel(
        _sc_deg_body,
        out_type=jax.ShapeDtypeStruct((NW, 1, 3 * NPAD), jnp.float32),
        mesh=mesh,
        compiler_params=params,
        scratch_types=[
            pltpu.VMEM((EW,), jnp.int32),
            pltpu.VMEM((NPAD,), jnp.float32),
            pltpu.VMEM((NPAD,), jnp.float32),
            pltpu.VMEM((NPAD,), jnp.float32),
        ],
    )
    return sc_pass1, sc_deg


def _segment_partials(y, src_w, dst_w, zeros_h):
    return _build_sc_kernels()[0](y, src_w, dst_w, zeros_h)


def _deg_partials(dst0_w, dst1_w, dst2_w, zeros1):
    return _build_sc_kernels()[1](dst0_w, dst1_w, dst2_w, zeros1)


def _full(shape):
    return pl.BlockSpec(shape, lambda i: (0,) * len(shape))


def _lstm_flat(x_ref, WiT, WhT, b):
    h = jnp.zeros((BN, HH), jnp.float32)
    c = jnp.zeros((BN, HH), jnp.float32)
    for t in range(T):
        xt = x_ref[:, t * DV:(t + 1) * DV]
        g = jnp.dot(xt, WiT, preferred_element_type=jnp.float32)
        g = g + jnp.dot(h, WhT, preferred_element_type=jnp.float32) + b
        i = jax.nn.sigmoid(g[:, 0 * HH:1 * HH])
        f = jax.nn.sigmoid(g[:, 1 * HH:2 * HH])
        gg = jnp.tanh(g[:, 2 * HH:3 * HH])
        o = jax.nn.sigmoid(g[:, 3 * HH:4 * HH])
        c = f * c + i * gg
        h = o * jnp.tanh(c)
    return h


def _leaky(x):
    return jnp.where(x >= 0, x, 0.01 * x)


def _encoder_body(xvf_ref, xsf_ref, per_ref,
                  WivT, WhvT, bv, WisT, WhsT, bs,
                  LvT, lbv, LsT, lbs, LpT, lbp,
                  W2T, b2, W3T, b3, W4T, b4,
                  x0_o, x1_o, x2_o):
    hv = _lstm_flat(xvf_ref, WivT[:], WhvT[:], bv[:])
    hs = _lstm_flat(xsf_ref, WisT[:], WhsT[:], bs[:])
    xv = jnp.dot(hv, LvT[:], preferred_element_type=jnp.float32) + lbv[:]
    xs = jnp.dot(hs, LsT[:], preferred_element_type=jnp.float32) + lbs[:]
    xp = jnp.dot(per_ref[:], LpT[:], preferred_element_type=jnp.float32) + lbp[:]
    xv2 = _leaky(jnp.concatenate([xv, xp], axis=1))
    xs2 = _leaky(jnp.concatenate([xs, xp], axis=1))
    x0_o[:] = _leaky(jnp.dot(xv2, W2T[:], preferred_element_type=jnp.float32)
                     + b2[:])
    x1_o[:] = _leaky(jnp.dot(xs2, W3T[:], preferred_element_type=jnp.float32)
                     + b3[:])
    xvs = jnp.concatenate([xv2, xs2], axis=1)
    x2_o[:] = _leaky(jnp.dot(xvs, W4T[:], preferred_element_type=jnp.float32)
                     + b4[:])


def _encoder(xvf, xsf, per, weights):
    in_specs = [
        pl.BlockSpec((BN, T * DV), lambda i: (i, 0)),
        pl.BlockSpec((BN, T * DS), lambda i: (i, 0)),
        pl.BlockSpec((BN, DP), lambda i: (i, 0)),
    ] + [_full(w.shape) for w in weights]
    return pl.pallas_call(
        _encoder_body,
        grid=(GRID,),
        in_specs=in_specs,
        out_specs=[pl.BlockSpec((BN, H), lambda i: (i, 0))] * 3,
        out_shape=[jax.ShapeDtypeStruct((N, H), jnp.float32)] * 3,
    )(xvf, xsf, per, *weights)


def _prep_body(degp_ref, x0_ref, x1_ref, x2_ref,
               dinv_o, y0_o, y1_o, y2_o):
    deg3 = jnp.sum(degp_ref[:], axis=0)
    xins = (x0_ref, x1_ref, x2_ref)
    youts = (y0_o, y1_o, y2_o)
    for r in range(3):
        dinv = lax.rsqrt(jnp.maximum(deg3[r], 1.0)).reshape(BN, 1)
        dinv_o[r] = dinv
        youts[r][:] = xins[r][:] * dinv


def _prep(degp, x0, x1, x2):
    outs = ([jax.ShapeDtypeStruct((3, N, 1), jnp.float32)]
            + [jax.ShapeDtypeStruct((N, H), jnp.float32)] * 3)
    out_specs = ([pl.BlockSpec((3, BN, 1), lambda i: (0, i, 0))]
                 + [pl.BlockSpec((BN, H), lambda i: (i, 0))] * 3)
    return pl.pallas_call(
        _prep_body,
        grid=(GRID,),
        in_specs=[pl.BlockSpec((NW, 3, BN), lambda i: (0, 0, i))]
        + [pl.BlockSpec((BN, H), lambda i: (i, 0))] * 3,
        out_specs=out_specs,
        out_shape=outs,
    )(degp, x0, x1, x2)


def _update_body(x_ref, p_ref, dinv_ref, f_o, y_o):
    agg = p_ref[0, 0] + p_ref[1, 0]
    dinv = dinv_ref[0]
    f = x_ref[:] - agg * dinv
    f_o[:] = f
    y_o[:] = f * dinv


def _update(x, p, dinv_r):
    return pl.pallas_call(
        _update_body,
        grid=(GRID,),
        in_specs=[
            pl.BlockSpec((BN, H), lambda i: (i, 0)),
            pl.BlockSpec((NC, 1, BN, H), lambda i: (0, 0, i, 0)),
            pl.BlockSpec((1, BN, 1), lambda i: (0, i, 0)),
        ],
        out_specs=[pl.BlockSpec((BN, H), lambda i: (i, 0))] * 2,
        out_shape=[jax.ShapeDtypeStruct((N, H), jnp.float32)] * 2,
    )(x, p, dinv_r)


_THETA_W = (
    (0.8, -0.5, 0.0),
    (3.0, -3.0, 0.75),
    (0.0, 3.0, -1.5),
    (0.0, 0.0, 0.75),
    (-0.2, 0.5, 0.0),
)


def _attn_one(f0, f1, f2, Wf1T, bf1, wf2, W5T, b5):
    hs = [t0 * f0 + t1 * f1 + t2 * f2 for (t0, t1, t2) in _THETA_W]
    ps = []
    for hk in hs:
        sk = jnp.tanh(jnp.dot(hk, Wf1T, preferred_element_type=jnp.float32)
                      + bf1)
        ps.append(jnp.sum(sk * wf2, axis=1, keepdims=True))
    m = ps[0]
    for pk in ps[1:]:
        m = jnp.maximum(m, pk)
    es = [jnp.exp(pk - m) for pk in ps]
    z = es[0]
    for ek in es[1:]:
        z = z + ek
    inv_z = 1.0 / z
    res = jnp.zeros(f0.shape, jnp.float32)
    for ek, hk in zip(es, hs):
        res = res + (ek * inv_z) * hk
    return jnp.dot(res, W5T, preferred_element_type=jnp.float32) + b5


def _tail_body(x0_ref, x1_ref, x2_ref, f10_ref, f11_ref, f12_ref,
               p0_ref, p1_ref, p2_ref, dinv_ref,
               Wf1T0, bf10, wf20, W5T0, b50,
               Wf1T1, bf11, wf21, W5T1, b51,
               Wf1T2, bf12, wf22, W5T2, b52,
               W6T, b6, out_o):
    xins = (x0_ref, x1_ref, x2_ref)
    f1s = (f10_ref, f11_ref, f12_ref)
    attw = ((Wf1T0, bf10, wf20, W5T0, b50),
            (Wf1T1, bf11, wf21, W5T1, b51),
            (Wf1T2, bf12, wf22, W5T2, b52))
    hs = []
    prefs = (p0_ref, p1_ref, p2_ref)
    for r in range(3):
        agg = prefs[r][0, 0] + prefs[r][1, 0]
        f2 = f1s[r][:] - agg * dinv_ref[r]
        Wf1T, bf1, wf2, W5T, b5 = attw[r]
        hs.append(_attn_one(xins[r][:], f1s[r][:], f2,
                            Wf1T[:], bf1[:], wf2[:], W5T[:], b5[:]))
    hcat = _leaky(jnp.concatenate(hs, axis=1))
    full = jnp.concatenate([hcat, xins[0][:], xins[1][:], xins[2][:]], axis=1)
    out_o[:] = jnp.dot(full, W6T[:], preferred_element_type=jnp.float32) + b6[:]


def _tail(x0, x1, x2, f10, f11, f12, p20, p21, p22, dinv, att_w, W6T, b6):
    flat_w = [w for ws in att_w for w in ws] + [W6T, b6]
    return pl.pallas_call(
        _tail_body,
        grid=(GRID,),
        in_specs=[pl.BlockSpec((BN, H), lambda i: (i, 0))] * 6
        + [pl.BlockSpec((NC, 1, BN, H), lambda i: (0, 0, i, 0))] * 3
        + [pl.BlockSpec((3, BN, 1), lambda i: (0, i, 0))]
        + [_full(w.shape) for w in flat_w],
        out_specs=pl.BlockSpec((BN, 128), lambda i: (i, 0)),
        out_shape=jax.ShapeDtypeStruct((N, 128), jnp.float32),
    )(x0, x1, x2, f10, f11, f12, p20, p21, p22, dinv, *flat_w)


def kernel(voc_features, sms_features, personal_feature,
           edge_index0, edge_index1, edge_index2,
           lstm_voc_Wih, lstm_voc_Whh, lstm_voc_bih, lstm_voc_bhh,
           lstm_sms_Wih, lstm_sms_Whh, lstm_sms_bih, lstm_sms_bhh,
           lin_voc_W, lin_voc_b, lin_sms_W, lin_sms_b, lin_per_W, lin_per_b,
           lin2_W, lin2_b, lin3_W, lin3_b, lin4_W, lin4_b,
           Wf1_0, bf1_0, Wf2_0, Wf1_1, bf1_1, Wf2_1, Wf1_2, bf1_2, Wf2_2,
           lin5_W0, lin5_b0, lin5_W1, lin5_b1, lin5_W2, lin5_b2,
           lin6_W, lin6_b):
    f32 = jnp.float32
    xvf = voc_features.reshape(N, T * DV)
    xsf = sms_features.reshape(N, T * DS)
    per = personal_feature
    srcs, dsts, dsts_flat = [], [], []
    for e in (edge_index0, edge_index1, edge_index2):
        srcs.append(e[0].reshape(NW, NCHUNK, CH))
        dsts.append(e[1].reshape(NW, NCHUNK, CH))
        dsts_flat.append(e[1].reshape(NW, 1, EW))
    zeros_h = jnp.zeros((NPAD, H), f32)
    zeros1 = jnp.zeros((NPAD,), f32)

    enc_w = [
        lstm_voc_Wih.T, lstm_voc_Whh.T,
        (lstm_voc_bih + lstm_voc_bhh).reshape(1, 4 * HH),
        lstm_sms_Wih.T, lstm_sms_Whh.T,
        (lstm_sms_bih + lstm_sms_bhh).reshape(1, 4 * HH),
        lin_voc_W.T, lin_voc_b.reshape(1, H),
        lin_sms_W.T, lin_sms_b.reshape(1, H),
        lin_per_W.T, lin_per_b.reshape(1, H),
        lin2_W.T, lin2_b.reshape(1, H),
        lin3_W.T, lin3_b.reshape(1, H),
        lin4_W.T, lin4_b.reshape(1, H),
    ]

    degp = _deg_partials(dsts_flat[0], dsts_flat[1], dsts_flat[2], zeros1)
    degp = degp.reshape(NW, 3, NPAD)
    x0, x1, x2 = _encoder(xvf, xsf, per, enc_w)
    dinv, y00, y01, y02 = _prep(degp, x0, x1, x2)

    att_w = (
        (Wf1_0.T, bf1_0.reshape(1, H), Wf2_0, lin5_W0.T, lin5_b0.reshape(1, H)),
        (Wf1_1.T, bf1_1.reshape(1, H), Wf2_1, lin5_W1.T, lin5_b1.reshape(1, H)),
        (Wf1_2.T, bf1_2.reshape(1, H), Wf2_2, lin5_W2.T, lin5_b2.reshape(1, H)),
    )

    xs = (x0, x1, x2)
    y0s = (y00, y01, y02)
    f1s, p2s = [], []
    for r in range(3):
        p1r = _segment_partials(y0s[r], srcs[r], dsts[r], zeros_h)
        fr, zr = _update(xs[r], p1r, dinv[r:r + 1])
        f1s.append(fr)
        p2s.append(_segment_partials(zr, srcs[r], dsts[r], zeros_h))

    W6T = jnp.zeros((6 * H, 128), f32).at[:, :C].set(lin6_W.T)
    b6 = jnp.zeros((1, 128), f32).at[0, :C].set(lin6_b)
    out = _tail(x0, x1, x2, f1s[0], f1s[1], f1s[2],
                p2s[0], p2s[1], p2s[2], dinv, att_w, W6T, b6)
    return out[:, :C]
